# baseline (speedup 1.0000x reference)
import functools

import jax
import jax.numpy as jnp
from jax import lax
from jax.experimental import pallas as pl
from jax.experimental.pallas import tpu as pltpu
from jax.experimental.pallas import tpu_sc as plsc

_B = 4096
_QL = 20
_DL = 200
_EMBED = 64
_NC = 2
_NS = 16
_NW = _NC * _NS
_RPW = _B // _NW
_DCH = 4
_QCH = 16


def _sum_rows(buf_v, base, n, outb_v, out_row):
    def tok(j, accs):
        return tuple(
            accs[c] + buf_v[base + j, pl.ds(16 * c, 16)] for c in range(4)
        )
    accs = lax.fori_loop(
        0, n, tok, tuple(jnp.zeros((16,), jnp.float32) for _ in range(4))
    )
    for c in range(4):
        outb_v[out_row, pl.ds(16 * c, 16)] = accs[c]


def _pool_body(qs_ref, ds_ref, qt_ref, dt_ref, qo_ref, do_ref,
               qidx_v, qbuf_v, didx_v, dbuf_v, outb_v, sem):
    wid = lax.axis_index("s") * _NC + lax.axis_index("c")

    def run_phase(idx_hbm, tab_hbm, out_hbm, seq_len, ch_rows, idx_v, buf_v):
        k = ch_rows * seq_len
        nch = _RPW // ch_rows
        base = wid * _RPW * seq_len

        def chunk(ch, carry):
            pltpu.sync_copy(idx_hbm.at[pl.ds(base + ch * k, k)], idx_v)
            pltpu.async_copy(tab_hbm.at[idx_v], buf_v, sem).wait()
            for r in range(ch_rows):
                _sum_rows(buf_v, r * seq_len, seq_len, outb_v,
                          ch * ch_rows + r)
            return carry

        lax.fori_loop(0, nch, chunk, 0)
        pltpu.sync_copy(outb_v, out_hbm.at[pl.ds(wid * _RPW, _RPW)])

    run_phase(qs_ref, qt_ref, qo_ref, _QL, _QCH, qidx_v, qbuf_v)
    run_phase(ds_ref, dt_ref, do_ref, _DL, _DCH, didx_v, dbuf_v)


def _sc_pool(qs_flat, ds_flat, q_table, d_table):
    mesh = plsc.VectorSubcoreMesh(core_axis_name="c", subcore_axis_name="s")
    out_type = (
        jax.ShapeDtypeStruct((_B, _EMBED), jnp.float32),
        jax.ShapeDtypeStruct((_B, _EMBED), jnp.float32),
    )
    scratch = [
        pltpu.VMEM((_QCH * _QL,), jnp.int32),
        pltpu.VMEM((_QCH * _QL, _EMBED), jnp.float32),
        pltpu.VMEM((_DCH * _DL,), jnp.int32),
        pltpu.VMEM((_DCH * _DL, _EMBED), jnp.float32),
        pltpu.VMEM((_RPW, _EMBED), jnp.float32),
        pltpu.SemaphoreType.DMA,
    ]
    f = pl.kernel(_pool_body, out_type=out_type, mesh=mesh,
                  scratch_types=scratch,
                  compiler_params=pltpu.CompilerParams(
                      use_tc_tiling_on_sc=False))
    return f(qs_flat, ds_flat, q_table, d_table)


def _epilogue_body(qs_ref, ds_ref, o_ref):
    q = jnp.tanh(qs_ref[...] * (1.0 / _QL))
    d = jnp.tanh(ds_ref[...] * (1.0 / _DL))
    qn = jnp.sqrt(jnp.sum(q * q, axis=1, keepdims=True))
    dn = jnp.sqrt(jnp.sum(d * d, axis=1, keepdims=True))
    q = q / jnp.maximum(qn, 1e-12)
    d = d / jnp.maximum(dn, 1e-12)
    o_ref[...] = jnp.sum(q * d, axis=1)


def _tc_epilogue(q_sum, d_sum):
    return pl.pallas_call(
        _epilogue_body,
        out_shape=jax.ShapeDtypeStruct((_B,), jnp.float32),
    )(q_sum, d_sum)


def kernel(qs, ds, rels, q_table, d_table):
    del rels
    q_sum, d_sum = _sc_pool(
        qs.reshape(-1), ds.reshape(-1), q_table, d_table
    )
    return _tc_epilogue(q_sum, d_sum)
